# baseline (speedup 1.0000x reference)
import functools

import jax
import jax.numpy as jnp
from jax import lax
from jax.experimental import pallas as pl
from jax.experimental.pallas import tpu as pltpu
from jax.experimental.pallas import tpu_sc as plsc

N = 10000
E = 320000
IN_F = 128
HID = 256
OUT_F = 128
D = 128

NC = 2
NS = 16
NW = NC * NS
CH = 128
K = 80
KB = 8
E_PAD = NW * CH * K
N_PAD = 10240
RPT = N_PAD // NS
JUNK_ROW = N


def _make_segsum(with_deg):
  mesh = plsc.VectorSubcoreMesh(core_axis_name="c", subcore_axis_name="s")
  out_type = [jax.ShapeDtypeStruct((NC, N_PAD, D), jnp.float32)]
  if with_deg:
    out_type.append(jax.ShapeDtypeStruct((NC, N_PAD // 128, 128), jnp.float32))

  @functools.partial(
      pl.kernel,
      mesh=mesh,
      out_type=out_type,
      compiler_params=pltpu.CompilerParams(needs_layout_passes=False),
      scratch_types=[
          pltpu.VMEM((KB, CH), jnp.int32),
          pltpu.VMEM((KB, CH), jnp.int32),
          pltpu.VMEM((KB, CH), jnp.int32),
          pltpu.VMEM((KB, CH), jnp.int32),
          pltpu.VMEM((CH, D), jnp.float32),
          pltpu.VMEM((CH, D), jnp.float32),
          pltpu.VMEM((N_PAD // 128, 128), jnp.float32),
          pltpu.VMEM_SHARED((N_PAD, D), jnp.float32),
          pltpu.SemaphoreType.DMA,
          pltpu.SemaphoreType.DMA,
          pltpu.SemaphoreType.DMA,
      ],
  )
  def segsum(table_hbm, eidx_hbm, *rest):
    if with_deg:
      (out_hbm, deg_hbm, src_v0, dst_v0, src_v1, dst_v1, rows0, rows1,
       deg2_v, acc, sem_g, sem_i, sem_s) = rest
    else:
      (out_hbm, src_v0, dst_v0, src_v1, dst_v1, rows0, rows1,
       deg2_v, acc, sem_g, sem_i, sem_s) = rest
    c = lax.axis_index("c")
    s = lax.axis_index("s")
    wid = s * NC + c
    idx_bufs = [(src_v0, dst_v0), (src_v1, dst_v1)]
    row_bufs = [rows0, rows1]

    zeros16 = jnp.zeros((16,), jnp.float32)
    ones16 = jnp.ones((16,), jnp.float32)

    def zrow(r, _):
      def zcol(j, __):
        rows0[r, pl.ds(j * 16, 16)] = zeros16
        return 0
      return lax.fori_loop(0, D // 16, zcol, 0)

    lax.fori_loop(0, CH, zrow, 0)

    def zacc(b, _):
      pltpu.sync_copy(rows0, acc.at[pl.ds(s * RPT + b * CH, CH)])
      return 0

    lax.fori_loop(0, RPT // CH, zacc, 0)

    if with_deg:
      def zdeg(r, _):
        def zdc(j, __):
          deg2_v[r, pl.ds(j * 16, 16)] = zeros16
          return 0
        return lax.fori_loop(0, 128 // 16, zdc, 0)
      lax.fori_loop(0, N_PAD // 128, zdeg, 0)

    plsc.subcore_barrier()

    def hist(dv, j):
      if with_deg:
        def h(t, __):
          idx16 = dv[j, pl.ds(t * 16, 16)]
          plsc.addupdate_scatter(deg2_v, [idx16 >> 7, idx16 & 127], ones16)
          return 0
        lax.fori_loop(0, CH // 16, h, 0)

    def gather(sv, j, rows):
      pltpu.async_copy(table_hbm.at[sv.at[j]], rows, sem_g)

    def gwait(rows):
      pltpu.make_async_copy(table_hbm.at[src_v0.at[0]], rows, sem_g).wait()

    def sstart(dv, j, rows):
      pltpu.async_copy(rows, acc.at[dv.at[j]], sem_s, add=True)

    def swait():
      pltpu.make_async_copy(rows0, acc.at[dst_v0.at[0]], sem_s).wait()

    NBLK = K // KB
    pltpu.sync_copy(eidx_hbm.at[0].at[wid].at[pl.ds(0, KB)], src_v0)
    pltpu.sync_copy(eidx_hbm.at[1].at[wid].at[pl.ds(0, KB)], dst_v0)
    gather(src_v0, 0, rows0)

    for b in range(NBLK):
      sv, dv = idx_bufs[b % 2]
      nsv, ndv = idx_bufs[(b + 1) % 2]
      if b + 1 < NBLK:
        pltpu.async_copy(eidx_hbm.at[0].at[wid].at[pl.ds((b + 1) * KB, KB)],
                         nsv, sem_i)
        pltpu.async_copy(eidx_hbm.at[1].at[wid].at[pl.ds((b + 1) * KB, KB)],
                         ndv, sem_i)

      def pair(t, _):
        gwait(rows0)
        if b == 0:
          @pl.when(t > 0)
          def _w0():
            swait()
        else:
          swait()
        gather(sv, 2 * t + 1, rows1)
        hist(dv, 2 * t)
        sstart(dv, 2 * t, rows0)
        gwait(rows1)
        swait()
        gather(sv, 2 * t + 2, rows0)
        hist(dv, 2 * t + 1)
        sstart(dv, 2 * t + 1, rows1)
        return 0

      lax.fori_loop(0, KB // 2 - 1, pair, 0)

      gwait(rows0)
      swait()
      gather(sv, KB - 1, rows1)
      hist(dv, KB - 2)
      sstart(dv, KB - 2, rows0)
      gwait(rows1)
      swait()
      if b + 1 < NBLK:
        pltpu.make_async_copy(
            eidx_hbm.at[0].at[wid].at[pl.ds(0, KB)], nsv, sem_i).wait()
        pltpu.make_async_copy(
            eidx_hbm.at[1].at[wid].at[pl.ds(0, KB)], ndv, sem_i).wait()
        gather(nsv, 0, rows0)
      hist(dv, KB - 1)
      sstart(dv, KB - 1, rows1)

    swait()
    plsc.subcore_barrier()

    def wout(b, _):
      sl = pl.ds(s * RPT + b * CH, CH)
      pltpu.sync_copy(acc.at[sl], out_hbm.at[c].at[sl])
      return 0

    lax.fori_loop(0, RPT // CH, wout, 0)

    if with_deg:
      plsc.subcore_barrier()
      pltpu.sync_copy(deg2_v, acc.at[pl.ds(s * (N_PAD // 128), N_PAD // 128)])
      plsc.subcore_barrier()

      @pl.when(s < N_PAD // 128 // 8)
      def _combine():
        pltpu.sync_copy(acc.at[pl.ds(s * 8, 8)], rows0.at[pl.ds(0, 8)])

        def comb(t, _):
          pltpu.sync_copy(acc.at[pl.ds(t * (N_PAD // 128) + s * 8, 8)],
                          rows1.at[pl.ds(0, 8)])
          def addv(r, __):
            def addc(j, ___):
              sl = pl.ds(j * 16, 16)
              rows0[r, sl] = rows0[r, sl] + rows1[r, sl]
              return 0
            return lax.fori_loop(0, 128 // 16, addc, 0)
          lax.fori_loop(0, 8, addv, 0)
          return 0

        lax.fori_loop(1, NS, comb, 0)
        pltpu.sync_copy(rows0.at[pl.ds(0, 8)], deg_hbm.at[c].at[pl.ds(s * 8, 8)])

  return segsum


_segsum_deg = _make_segsum(True)
_segsum_nodeg = _make_segsum(False)


BR = 400


def _layer1_body(x_ref, p0_ref, p1_ref, d0_ref, d1_ref, ws1_ref, wn1_ref,
                 b1_ref, ws2_ref, wn2_ref, b2_ref, yn_ref, ys_ref, inv_ref):
  deg = d0_ref[...] + d1_ref[...]
  inv = 1.0 / jnp.maximum(deg, 1.0)
  mean = (p0_ref[...] + p1_ref[...]) * inv
  h = (jnp.dot(x_ref[...], ws1_ref[...], preferred_element_type=jnp.float32)
       + jnp.dot(mean, wn1_ref[...], preferred_element_type=jnp.float32)
       + b1_ref[...])
  h = jnp.maximum(h, 0.0)
  yn_ref[...] = jnp.dot(h, wn2_ref[...], preferred_element_type=jnp.float32)
  ys_ref[...] = (jnp.dot(h, ws2_ref[...], preferred_element_type=jnp.float32)
                 + b2_ref[...])
  inv_ref[...] = jnp.broadcast_to(inv, (BR, OUT_F))


def _tc_layer1(x, p0, p1, d0, d1, ws1, wn1, b1, ws2, wn2, b2):
  return pl.pallas_call(
      _layer1_body,
      grid=(N // BR,),
      in_specs=[
          pl.BlockSpec((BR, IN_F), lambda i: (i, 0)),
          pl.BlockSpec((BR, D), lambda i: (i, 0)),
          pl.BlockSpec((BR, D), lambda i: (i, 0)),
          pl.BlockSpec((BR, 1), lambda i: (i, 0)),
          pl.BlockSpec((BR, 1), lambda i: (i, 0)),
          pl.BlockSpec((IN_F, HID), lambda i: (0, 0)),
          pl.BlockSpec((IN_F, HID), lambda i: (0, 0)),
          pl.BlockSpec((1, HID), lambda i: (0, 0)),
          pl.BlockSpec((HID, OUT_F), lambda i: (0, 0)),
          pl.BlockSpec((HID, OUT_F), lambda i: (0, 0)),
          pl.BlockSpec((1, OUT_F), lambda i: (0, 0)),
      ],
      out_specs=[
          pl.BlockSpec((BR, OUT_F), lambda i: (i, 0)),
          pl.BlockSpec((BR, OUT_F), lambda i: (i, 0)),
          pl.BlockSpec((BR, OUT_F), lambda i: (i, 0)),
      ],
      out_shape=[
          jax.ShapeDtypeStruct((N, OUT_F), jnp.float32),
          jax.ShapeDtypeStruct((N, OUT_F), jnp.float32),
          jax.ShapeDtypeStruct((N, OUT_F), jnp.float32),
      ],
  )(x, p0, p1, d0, d1, ws1, wn1, b1, ws2, wn2, b2)


def _final_body(ys_ref, q0_ref, q1_ref, inv_ref, o_ref):
  o_ref[...] = ys_ref[...] + (q0_ref[...] + q1_ref[...]) * inv_ref[...]


def _tc_final(ys, q0, q1, inv):
  return pl.pallas_call(
      _final_body,
      grid=(N // BR,),
      in_specs=[
          pl.BlockSpec((BR, OUT_F), lambda i: (i, 0)),
          pl.BlockSpec((BR, D), lambda i: (i, 0)),
          pl.BlockSpec((BR, D), lambda i: (i, 0)),
          pl.BlockSpec((BR, OUT_F), lambda i: (i, 0)),
      ],
      out_specs=pl.BlockSpec((BR, OUT_F), lambda i: (i, 0)),
      out_shape=jax.ShapeDtypeStruct((N, OUT_F), jnp.float32),
  )(ys, q0, q1, inv)


def kernel(features, edge_index, W_self1, W_neigh1, b1, W_self2, W_neigh2, b2):
  src = edge_index[0].astype(jnp.int32)
  dst = edge_index[1].astype(jnp.int32)
  pad = E_PAD - E
  pad_ar = jnp.arange(pad, dtype=jnp.int32)
  eidx_p = jnp.stack([
      jnp.concatenate([src, pad_ar % N]),
      jnp.concatenate([dst, JUNK_ROW + pad_ar % (N_PAD - N)]),
  ]).reshape(2, NW, K, CH)

  p, degs = _segsum_deg(features, eidx_p)
  d0 = degs[0].reshape(N_PAD, 1)
  d1 = degs[1].reshape(N_PAD, 1)

  yn, ys, inv = _tc_layer1(features, p[0], p[1], d0, d1, W_self1, W_neigh1,
                           b1.reshape(1, HID), W_self2, W_neigh2,
                           b2.reshape(1, OUT_F))

  (q,) = _segsum_nodeg(yn, eidx_p)
  return _tc_final(ys, q[0], q[1], inv)


# speedup vs baseline: 1.1058x; 1.1058x over previous
import functools

import jax
import jax.numpy as jnp
from jax import lax
from jax.experimental import pallas as pl
from jax.experimental.pallas import tpu as pltpu
from jax.experimental.pallas import tpu_sc as plsc

N = 10000
E = 320000
IN_F = 128
HID = 256
OUT_F = 128
D = 128

NC = 2
NS = 16
NW = NC * NS
CH = 128
K = 80
KB = 8
E_PAD = NW * CH * K
N_PAD = 10240
RPT = N_PAD // NS
JUNK_ROW = N


def _make_segsum(with_deg):
  mesh = plsc.VectorSubcoreMesh(core_axis_name="c", subcore_axis_name="s")
  out_type = [jax.ShapeDtypeStruct((N_PAD, D), jnp.float32),
              jax.ShapeDtypeStruct((N_PAD, D), jnp.float32)]
  if with_deg:
    out_type.append(jax.ShapeDtypeStruct((NC, N_PAD // 128, 128), jnp.float32))

  @functools.partial(
      pl.kernel,
      mesh=mesh,
      out_type=out_type,
      compiler_params=pltpu.CompilerParams(needs_layout_passes=False),
      scratch_types=[
          pltpu.VMEM((KB, CH), jnp.int32),
          pltpu.VMEM((KB, CH), jnp.int32),
          pltpu.VMEM((KB, CH), jnp.int32),
          pltpu.VMEM((KB, CH), jnp.int32),
          pltpu.VMEM((CH, D), jnp.float32),
          pltpu.VMEM((CH, D), jnp.float32),
          pltpu.VMEM((N_PAD // 128, 128), jnp.float32),
          pltpu.VMEM_SHARED((N_PAD, D), jnp.float32),
          pltpu.SemaphoreType.DMA,
          pltpu.SemaphoreType.DMA,
          pltpu.SemaphoreType.DMA,
      ],
  )
  def segsum(table_hbm, eidx_hbm, *rest):
    if with_deg:
      (out0_hbm, out1_hbm, deg_hbm, src_v0, dst_v0, src_v1, dst_v1, rows0,
       rows1, deg2_v, acc, sem_g, sem_i, sem_s) = rest
    else:
      (out0_hbm, out1_hbm, src_v0, dst_v0, src_v1, dst_v1, rows0, rows1,
       deg2_v, acc, sem_g, sem_i, sem_s) = rest
    c = lax.axis_index("c")
    s = lax.axis_index("s")
    wid = s * NC + c
    idx_bufs = [(src_v0, dst_v0), (src_v1, dst_v1)]
    row_bufs = [rows0, rows1]

    zeros16 = jnp.zeros((16,), jnp.float32)
    ones16 = jnp.ones((16,), jnp.float32)

    def zrow(r, _):
      def zcol(j, __):
        rows0[r, pl.ds(j * 16, 16)] = zeros16
        return 0
      return lax.fori_loop(0, D // 16, zcol, 0)

    lax.fori_loop(0, CH, zrow, 0)

    def zacc(b, _):
      pltpu.sync_copy(rows0, acc.at[pl.ds(s * RPT + b * CH, CH)])
      return 0

    lax.fori_loop(0, RPT // CH, zacc, 0)

    if with_deg:
      def zdeg(r, _):
        def zdc(j, __):
          deg2_v[r, pl.ds(j * 16, 16)] = zeros16
          return 0
        return lax.fori_loop(0, 128 // 16, zdc, 0)
      lax.fori_loop(0, N_PAD // 128, zdeg, 0)

    plsc.subcore_barrier()

    def hist(dv, j):
      if with_deg:
        def h(t, __):
          idx16 = dv[j, pl.ds(t * 16, 16)]
          plsc.addupdate_scatter(deg2_v, [idx16 >> 7, idx16 & 127], ones16)
          return 0
        lax.fori_loop(0, CH // 16, h, 0)

    def gather(sv, j, rows):
      pltpu.async_copy(table_hbm.at[sv.at[j]], rows, sem_g)

    def gwait(rows):
      pltpu.make_async_copy(table_hbm.at[src_v0.at[0]], rows, sem_g).wait()

    def sstart(dv, j, rows):
      pltpu.async_copy(rows, acc.at[dv.at[j]], sem_s, add=True)

    def swait():
      pltpu.make_async_copy(rows0, acc.at[dst_v0.at[0]], sem_s).wait()

    NBLK = K // KB
    pltpu.sync_copy(eidx_hbm.at[0].at[wid].at[pl.ds(0, KB)], src_v0)
    pltpu.sync_copy(eidx_hbm.at[1].at[wid].at[pl.ds(0, KB)], dst_v0)
    gather(src_v0, 0, rows0)

    for b in range(NBLK):
      sv, dv = idx_bufs[b % 2]
      nsv, ndv = idx_bufs[(b + 1) % 2]
      if b + 1 < NBLK:
        pltpu.async_copy(eidx_hbm.at[0].at[wid].at[pl.ds((b + 1) * KB, KB)],
                         nsv, sem_i)
        pltpu.async_copy(eidx_hbm.at[1].at[wid].at[pl.ds((b + 1) * KB, KB)],
                         ndv, sem_i)

      def pair(t, _):
        gwait(rows0)
        if b == 0:
          @pl.when(t > 0)
          def _w0():
            swait()
        else:
          swait()
        gather(sv, 2 * t + 1, rows1)
        hist(dv, 2 * t)
        sstart(dv, 2 * t, rows0)
        gwait(rows1)
        swait()
        gather(sv, 2 * t + 2, rows0)
        hist(dv, 2 * t + 1)
        sstart(dv, 2 * t + 1, rows1)
        return 0

      lax.fori_loop(0, KB // 2 - 1, pair, 0)

      gwait(rows0)
      swait()
      gather(sv, KB - 1, rows1)
      hist(dv, KB - 2)
      sstart(dv, KB - 2, rows0)
      gwait(rows1)
      swait()
      if b + 1 < NBLK:
        pltpu.make_async_copy(
            eidx_hbm.at[0].at[wid].at[pl.ds(0, KB)], nsv, sem_i).wait()
        pltpu.make_async_copy(
            eidx_hbm.at[1].at[wid].at[pl.ds(0, KB)], ndv, sem_i).wait()
        gather(nsv, 0, rows0)
      hist(dv, KB - 1)
      sstart(dv, KB - 1, rows1)

    swait()
    plsc.subcore_barrier()

    @pl.when(c == 0)
    def _w0():
      def wout(b, _):
        sl = pl.ds(s * RPT + b * CH, CH)
        pltpu.sync_copy(acc.at[sl], out0_hbm.at[sl])
        return 0
      lax.fori_loop(0, RPT // CH, wout, 0)

    @pl.when(c == 1)
    def _w1():
      def wout(b, _):
        sl = pl.ds(s * RPT + b * CH, CH)
        pltpu.sync_copy(acc.at[sl], out1_hbm.at[sl])
        return 0
      lax.fori_loop(0, RPT // CH, wout, 0)

    if with_deg:
      plsc.subcore_barrier()
      pltpu.sync_copy(deg2_v, acc.at[pl.ds(s * (N_PAD // 128), N_PAD // 128)])
      plsc.subcore_barrier()

      @pl.when(s < N_PAD // 128 // 8)
      def _combine():
        pltpu.sync_copy(acc.at[pl.ds(s * 8, 8)], rows0.at[pl.ds(0, 8)])

        def comb(t, _):
          pltpu.sync_copy(acc.at[pl.ds(t * (N_PAD // 128) + s * 8, 8)],
                          rows1.at[pl.ds(0, 8)])
          def addv(r, __):
            def addc(j, ___):
              sl = pl.ds(j * 16, 16)
              rows0[r, sl] = rows0[r, sl] + rows1[r, sl]
              return 0
            return lax.fori_loop(0, 128 // 16, addc, 0)
          lax.fori_loop(0, 8, addv, 0)
          return 0

        lax.fori_loop(1, NS, comb, 0)
        pltpu.sync_copy(rows0.at[pl.ds(0, 8)], deg_hbm.at[c].at[pl.ds(s * 8, 8)])

  return segsum


_segsum_deg = _make_segsum(True)
_segsum_nodeg = _make_segsum(False)


BR = 2000


def _layer1_body(x_ref, p0_ref, p1_ref, d0_ref, d1_ref, ws1_ref, wn1_ref,
                 b1_ref, ws2_ref, wn2_ref, b2_ref, yn_ref, ys_ref, inv_ref):
  deg = d0_ref[...] + d1_ref[...]
  inv = 1.0 / jnp.maximum(deg, 1.0)
  mean = (p0_ref[...] + p1_ref[...]) * inv
  h = (jnp.dot(x_ref[...], ws1_ref[...], preferred_element_type=jnp.float32)
       + jnp.dot(mean, wn1_ref[...], preferred_element_type=jnp.float32)
       + b1_ref[...])
  h = jnp.maximum(h, 0.0)
  yn_ref[...] = jnp.dot(h, wn2_ref[...], preferred_element_type=jnp.float32)
  ys_ref[...] = (jnp.dot(h, ws2_ref[...], preferred_element_type=jnp.float32)
                 + b2_ref[...])
  inv_ref[...] = jnp.broadcast_to(inv, (BR, OUT_F))


def _tc_layer1(x, p0, p1, d0, d1, ws1, wn1, b1, ws2, wn2, b2):
  return pl.pallas_call(
      _layer1_body,
      grid=(N // BR,),
      in_specs=[
          pl.BlockSpec((BR, IN_F), lambda i: (i, 0)),
          pl.BlockSpec((BR, D), lambda i: (i, 0)),
          pl.BlockSpec((BR, D), lambda i: (i, 0)),
          pl.BlockSpec((BR, 1), lambda i: (i, 0)),
          pl.BlockSpec((BR, 1), lambda i: (i, 0)),
          pl.BlockSpec((IN_F, HID), lambda i: (0, 0)),
          pl.BlockSpec((IN_F, HID), lambda i: (0, 0)),
          pl.BlockSpec((1, HID), lambda i: (0, 0)),
          pl.BlockSpec((HID, OUT_F), lambda i: (0, 0)),
          pl.BlockSpec((HID, OUT_F), lambda i: (0, 0)),
          pl.BlockSpec((1, OUT_F), lambda i: (0, 0)),
      ],
      out_specs=[
          pl.BlockSpec((BR, OUT_F), lambda i: (i, 0)),
          pl.BlockSpec((BR, OUT_F), lambda i: (i, 0)),
          pl.BlockSpec((BR, OUT_F), lambda i: (i, 0)),
      ],
      out_shape=[
          jax.ShapeDtypeStruct((N, OUT_F), jnp.float32),
          jax.ShapeDtypeStruct((N, OUT_F), jnp.float32),
          jax.ShapeDtypeStruct((N, OUT_F), jnp.float32),
      ],
  )(x, p0, p1, d0, d1, ws1, wn1, b1, ws2, wn2, b2)


def _final_body(ys_ref, q0_ref, q1_ref, inv_ref, o_ref):
  o_ref[...] = ys_ref[...] + (q0_ref[...] + q1_ref[...]) * inv_ref[...]


def _tc_final(ys, q0, q1, inv):
  return pl.pallas_call(
      _final_body,
      grid=(N // BR,),
      in_specs=[
          pl.BlockSpec((BR, OUT_F), lambda i: (i, 0)),
          pl.BlockSpec((BR, D), lambda i: (i, 0)),
          pl.BlockSpec((BR, D), lambda i: (i, 0)),
          pl.BlockSpec((BR, OUT_F), lambda i: (i, 0)),
      ],
      out_specs=pl.BlockSpec((BR, OUT_F), lambda i: (i, 0)),
      out_shape=jax.ShapeDtypeStruct((N, OUT_F), jnp.float32),
  )(ys, q0, q1, inv)


def kernel(features, edge_index, W_self1, W_neigh1, b1, W_self2, W_neigh2, b2):
  src = edge_index[0].astype(jnp.int32)
  dst = edge_index[1].astype(jnp.int32)
  pad = E_PAD - E
  pad_ar = jnp.arange(pad, dtype=jnp.int32)
  eidx_p = jnp.stack([
      jnp.concatenate([src, pad_ar % N]),
      jnp.concatenate([dst, JUNK_ROW + pad_ar % (N_PAD - N)]),
  ]).reshape(2, NW, K, CH)

  p0, p1, degs = _segsum_deg(features, eidx_p)
  d0 = degs[0].reshape(N_PAD, 1)
  d1 = degs[1].reshape(N_PAD, 1)

  yn, ys, inv = _tc_layer1(features, p0, p1, d0, d1, W_self1, W_neigh1,
                           b1.reshape(1, HID), W_self2, W_neigh2,
                           b2.reshape(1, OUT_F))

  q0, q1 = _segsum_nodeg(yn, eidx_p)
  return _tc_final(ys, q0, q1, inv)
